# baseline (speedup 1.0000x reference)
import functools

import jax
import jax.numpy as jnp
from jax import lax
from jax.experimental import pallas as pl
from jax.experimental.pallas import tpu as pltpu
from jax.experimental.pallas import tpu_sc as plsc

N_NODES = 10000
N_EDGES = 320000
HIDDEN = 128

NC = 1
NS = 16
L = 16

EPT = N_EDGES // NS
S_PAD = 10240
ZSL = S_PAD // NS
PARAMS = 528
U = 10

_mesh = plsc.VectorSubcoreMesh(
    core_axis_name="c", subcore_axis_name="s", num_cores=NC)


@functools.partial(
    pl.kernel,
    out_type=jax.ShapeDtypeStruct((N_EDGES,), jnp.float32),
    mesh=_mesh,
    compiler_params=pltpu.CompilerParams(needs_layout_passes=False),
    scratch_types=[
        pltpu.VMEM((N_NODES,), jnp.float32),
        pltpu.VMEM((EPT,), jnp.int32),
        pltpu.VMEM((S_PAD,), jnp.float32),
        pltpu.VMEM((NS, ZSL), jnp.float32),
        pltpu.VMEM((EPT,), jnp.float32),
        pltpu.VMEM((PARAMS,), jnp.float32),
        pltpu.VMEM_SHARED((NS, S_PAD), jnp.float32),
        pltpu.VMEM_SHARED((S_PAD,), jnp.float32),
        pltpu.SemaphoreType.DMA,
        pltpu.SemaphoreType.DMA,
    ],
)
def _sc_kernel(x_hbm, ei_hbm, par_hbm, out_hbm,
               x_v, e_v, s_v, red_v, out_v, par_v,
               part_sh, s_sh, sem, sem2):
    s_id = lax.axis_index("s")

    d_par = pltpu.async_copy(par_hbm, par_v, sem2)
    d_x = pltpu.async_copy(x_hbm, x_v, sem)
    d_e = pltpu.async_copy(ei_hbm.at[pl.ds(s_id * EPT, EPT)], e_v, sem)

    zeros = jnp.zeros((L,), jnp.float32)

    @plsc.parallel_loop(0, S_PAD, L, unroll=8)
    def _(i):
        s_v[pl.ds(i, L)] = zeros

    d_x.wait()
    d_e.wait()

    @plsc.parallel_loop(0, EPT, L, unroll=U)
    def _(i):
        e = e_v[pl.ds(i, L)]
        idx_s = jnp.bitwise_and(e, 0xFFFF)
        idx_d = lax.shift_right_logical(e, 16)
        v = plsc.load_gather(x_v, [idx_s])
        plsc.addupdate_scatter(s_v, [idx_d], v)

    pltpu.sync_copy(s_v, part_sh.at[s_id])
    plsc.subcore_barrier()

    rd = []
    for t in range(NS):
        rd.append(pltpu.async_copy(
            part_sh.at[t, pl.ds(s_id * ZSL, ZSL)], red_v.at[t], sem))
    for d in rd:
        d.wait()

    @plsc.parallel_loop(0, ZSL, L, unroll=4)
    def _(k):
        acc = red_v[0, pl.ds(k, L)]
        for t in range(1, NS):
            acc = acc + red_v[t, pl.ds(k, L)]
        s_v[pl.ds(k, L)] = acc

    pltpu.sync_copy(s_v.at[pl.ds(0, ZSL)], s_sh.at[pl.ds(s_id * ZSL, ZSL)])
    plsc.subcore_barrier()

    d_s = pltpu.async_copy(s_sh, s_v, sem)

    d_par.wait()
    acc_a = jnp.zeros((L,), jnp.float32)
    acc_b = jnp.zeros((L,), jnp.float32)
    acc_c = jnp.zeros((L,), jnp.float32)
    for j in range(HIDDEN // L):
        w = par_v[pl.ds(j * L, L)]
        w1 = par_v[pl.ds(HIDDEN + j * L, L)]
        w2 = par_v[pl.ds(2 * HIDDEN + j * L, L)]
        bi = par_v[pl.ds(3 * HIDDEN + j * L, L)]
        acc_a = acc_a + w * w1
        acc_b = acc_b + w * w2
        acc_c = acc_c + bi * (w1 + w2)
    acc_c = acc_c + par_v[pl.ds(4 * HIDDEN, L)]

    lane = lax.iota(jnp.int32, L)
    _dnums = lax.GatherDimensionNumbers(
        offset_dims=(), collapsed_slice_dims=(0,), start_index_map=(0,))

    def lane_sum(v):
        for sh in (8, 4, 2, 1):
            perm = jnp.bitwise_xor(lane, sh)
            v = v + lax.gather(
                v, perm[:, None], dimension_numbers=_dnums, slice_sizes=(1,),
                mode=lax.GatherScatterMode.PROMISE_IN_BOUNDS)
        return v

    a = lane_sum(acc_a)
    b = lane_sum(acc_b)
    c = lane_sum(acc_c)

    d_s.wait()

    @plsc.parallel_loop(0, EPT, L, unroll=U)
    def _(i):
        e = e_v[pl.ds(i, L)]
        idx_s = jnp.bitwise_and(e, 0xFFFF)
        idx_d = lax.shift_right_logical(e, 16)
        vs = plsc.load_gather(s_v, [idx_s])
        vd = plsc.load_gather(s_v, [idx_d])
        out_v[pl.ds(i, L)] = a * vs + b * vd + c

    pltpu.sync_copy(out_v, out_hbm.at[pl.ds(s_id * EPT, EPT)])


def kernel(x, edge_index, W_lin, bias, W_dec, b_dec):
    ei32 = edge_index.astype(jnp.int32)
    ei = ei32[0] | (ei32[1] << 16)

    params = jnp.concatenate([
        W_lin.reshape(-1),
        W_dec.reshape(-1),
        bias.reshape(-1),
        jnp.pad(b_dec.reshape(-1), (0, L - 1)),
    ]).astype(jnp.float32)

    out_flat = _sc_kernel(x.reshape(-1), ei, params)
    return out_flat.reshape(N_EDGES, 1)


# speedup vs baseline: 1.2069x; 1.2069x over previous
import functools

import jax
import jax.numpy as jnp
from jax import lax
from jax.experimental import pallas as pl
from jax.experimental.pallas import tpu as pltpu
from jax.experimental.pallas import tpu_sc as plsc

N_NODES = 10000
N_EDGES = 320000
HIDDEN = 128

NC = 1
NS = 16
L = 16

EPT = N_EDGES // NS
S_PAD = 10240
ZSL = S_PAD // NS
PARAMS = 528
U = 10

_mesh = plsc.VectorSubcoreMesh(
    core_axis_name="c", subcore_axis_name="s", num_cores=NC)


@functools.partial(
    pl.kernel,
    out_type=jax.ShapeDtypeStruct((N_EDGES,), jnp.float32),
    mesh=_mesh,
    compiler_params=pltpu.CompilerParams(
        needs_layout_passes=False, use_tc_tiling_on_sc=False),
    scratch_types=[
        pltpu.VMEM((N_NODES,), jnp.float32),
        pltpu.VMEM((EPT,), jnp.int32),
        pltpu.VMEM((EPT,), jnp.int32),
        pltpu.VMEM((S_PAD,), jnp.float32),
        pltpu.VMEM((NS, ZSL), jnp.float32),
        pltpu.VMEM((EPT,), jnp.float32),
        pltpu.VMEM((PARAMS,), jnp.float32),
        pltpu.VMEM_SHARED((NS, S_PAD), jnp.float32),
        pltpu.VMEM_SHARED((S_PAD,), jnp.float32),
        pltpu.SemaphoreType.DMA,
        pltpu.SemaphoreType.DMA,
    ],
)
def _sc_kernel(x_hbm, ei_hbm, par_hbm, out_hbm,
               x_v, src_v, dst_v, s_v, red_v, out_v, par_v,
               part_sh, s_sh, sem, sem2):
    s_id = lax.axis_index("s")

    d_par = pltpu.async_copy(par_hbm, par_v, sem2)
    d_x = pltpu.async_copy(x_hbm, x_v, sem)
    d_src = pltpu.async_copy(ei_hbm.at[pl.ds(s_id * EPT, EPT)], src_v, sem)
    d_dst = pltpu.async_copy(
        ei_hbm.at[pl.ds(N_EDGES + s_id * EPT, EPT)], dst_v, sem)

    zeros = jnp.zeros((L,), jnp.float32)

    @plsc.parallel_loop(0, S_PAD, L, unroll=8)
    def _(i):
        s_v[pl.ds(i, L)] = zeros

    d_x.wait()
    d_src.wait()
    d_dst.wait()

    @plsc.parallel_loop(0, EPT, L, unroll=U)
    def _(i):
        idx_s = src_v[pl.ds(i, L)]
        idx_d = dst_v[pl.ds(i, L)]
        v = plsc.load_gather(x_v, [idx_s])
        plsc.addupdate_scatter(s_v, [idx_d], v)

    pltpu.sync_copy(s_v, part_sh.at[s_id])
    plsc.subcore_barrier()

    rd = []
    for t in range(NS):
        rd.append(pltpu.async_copy(
            part_sh.at[t, pl.ds(s_id * ZSL, ZSL)], red_v.at[t], sem))
    for d in rd:
        d.wait()

    @plsc.parallel_loop(0, ZSL, L, unroll=4)
    def _(k):
        acc = red_v[0, pl.ds(k, L)]
        for t in range(1, NS):
            acc = acc + red_v[t, pl.ds(k, L)]
        s_v[pl.ds(k, L)] = acc

    pltpu.sync_copy(s_v.at[pl.ds(0, ZSL)], s_sh.at[pl.ds(s_id * ZSL, ZSL)])
    plsc.subcore_barrier()

    d_s = pltpu.async_copy(s_sh, s_v, sem)

    d_par.wait()
    acc_a = jnp.zeros((L,), jnp.float32)
    acc_b = jnp.zeros((L,), jnp.float32)
    acc_c = jnp.zeros((L,), jnp.float32)
    for j in range(HIDDEN // L):
        w = par_v[pl.ds(j * L, L)]
        w1 = par_v[pl.ds(HIDDEN + j * L, L)]
        w2 = par_v[pl.ds(2 * HIDDEN + j * L, L)]
        bi = par_v[pl.ds(3 * HIDDEN + j * L, L)]
        acc_a = acc_a + w * w1
        acc_b = acc_b + w * w2
        acc_c = acc_c + bi * (w1 + w2)
    acc_c = acc_c + par_v[pl.ds(4 * HIDDEN, L)]

    lane = lax.iota(jnp.int32, L)
    _dnums = lax.GatherDimensionNumbers(
        offset_dims=(), collapsed_slice_dims=(0,), start_index_map=(0,))

    def lane_sum(v):
        for sh in (8, 4, 2, 1):
            perm = jnp.bitwise_xor(lane, sh)
            v = v + lax.gather(
                v, perm[:, None], dimension_numbers=_dnums, slice_sizes=(1,),
                mode=lax.GatherScatterMode.PROMISE_IN_BOUNDS)
        return v

    a = lane_sum(acc_a)
    b = lane_sum(acc_b)
    c = lane_sum(acc_c)

    d_s.wait()

    @plsc.parallel_loop(0, EPT, L, unroll=U)
    def _(i):
        vs = plsc.load_gather(s_v, [src_v[pl.ds(i, L)]])
        vd = plsc.load_gather(s_v, [dst_v[pl.ds(i, L)]])
        out_v[pl.ds(i, L)] = a * vs + b * vd + c

    pltpu.sync_copy(out_v, out_hbm.at[pl.ds(s_id * EPT, EPT)])


def kernel(x, edge_index, W_lin, bias, W_dec, b_dec):
    ei = edge_index.astype(jnp.int32).reshape(-1)

    params = jnp.concatenate([
        W_lin[0],
        W_dec[:, 0],
        bias,
        jnp.pad(b_dec, (0, L - 1)),
    ])

    return _sc_kernel(x[:, 0], ei, params).reshape(N_EDGES, 1)


# speedup vs baseline: 1.2100x; 1.0025x over previous
import functools

import jax
import jax.numpy as jnp
from jax import lax
from jax.experimental import pallas as pl
from jax.experimental.pallas import tpu as pltpu
from jax.experimental.pallas import tpu_sc as plsc

N_NODES = 10000
N_EDGES = 320000
HIDDEN = 128

NC = 1
NS = 16
L = 16

EPT = N_EDGES // NS
S_PAD = 10240
ZSL = S_PAD // NS
PARAMS = 528
U = 10

_mesh = plsc.VectorSubcoreMesh(
    core_axis_name="c", subcore_axis_name="s", num_cores=NC)


@functools.partial(
    pl.kernel,
    out_type=jax.ShapeDtypeStruct((N_EDGES,), jnp.float32),
    mesh=_mesh,
    compiler_params=pltpu.CompilerParams(
        needs_layout_passes=False, use_tc_tiling_on_sc=False),
    scratch_types=[
        pltpu.VMEM((N_NODES,), jnp.float32),
        pltpu.VMEM((EPT,), jnp.int32),
        pltpu.VMEM((EPT,), jnp.int32),
        pltpu.VMEM((S_PAD,), jnp.float32),
        pltpu.VMEM((NS, ZSL), jnp.float32),
        pltpu.VMEM((EPT,), jnp.float32),
        pltpu.VMEM((PARAMS,), jnp.float32),
        pltpu.VMEM_SHARED((NS, S_PAD), jnp.float32),
        pltpu.VMEM_SHARED((S_PAD,), jnp.float32),
        pltpu.SemaphoreType.DMA,
        pltpu.SemaphoreType.DMA,
    ],
)
def _sc_kernel(x_hbm, ei_hbm, par_hbm, out_hbm,
               x_v, src_v, dst_v, s_v, red_v, out_v, par_v,
               part_sh, s_sh, sem, sem2):
    s_id = lax.axis_index("s")

    d_par = pltpu.async_copy(par_hbm, par_v, sem2)
    d_x = pltpu.async_copy(x_hbm, x_v, sem)
    d_src = pltpu.async_copy(ei_hbm.at[0, pl.ds(s_id * EPT, EPT)], src_v, sem)
    d_dst = pltpu.async_copy(ei_hbm.at[1, pl.ds(s_id * EPT, EPT)], dst_v, sem)

    zeros = jnp.zeros((L,), jnp.float32)

    @plsc.parallel_loop(0, S_PAD, L, unroll=8)
    def _(i):
        s_v[pl.ds(i, L)] = zeros

    d_x.wait()
    d_src.wait()
    d_dst.wait()

    @plsc.parallel_loop(0, EPT, L, unroll=U)
    def _(i):
        idx_s = src_v[pl.ds(i, L)]
        idx_d = dst_v[pl.ds(i, L)]
        v = plsc.load_gather(x_v, [idx_s])
        plsc.addupdate_scatter(s_v, [idx_d], v)

    pltpu.sync_copy(s_v, part_sh.at[s_id])
    plsc.subcore_barrier()

    rd = []
    for t in range(NS):
        rd.append(pltpu.async_copy(
            part_sh.at[t, pl.ds(s_id * ZSL, ZSL)], red_v.at[t], sem))
    for d in rd:
        d.wait()

    @plsc.parallel_loop(0, ZSL, L, unroll=4)
    def _(k):
        acc = red_v[0, pl.ds(k, L)]
        for t in range(1, NS):
            acc = acc + red_v[t, pl.ds(k, L)]
        s_v[pl.ds(k, L)] = acc

    pltpu.sync_copy(s_v.at[pl.ds(0, ZSL)], s_sh.at[pl.ds(s_id * ZSL, ZSL)])
    plsc.subcore_barrier()

    d_s = pltpu.async_copy(s_sh, s_v, sem)

    d_par.wait()
    acc_a = jnp.zeros((L,), jnp.float32)
    acc_b = jnp.zeros((L,), jnp.float32)
    acc_c = jnp.zeros((L,), jnp.float32)
    for j in range(HIDDEN // L):
        w = par_v[pl.ds(j * L, L)]
        w1 = par_v[pl.ds(HIDDEN + j * L, L)]
        w2 = par_v[pl.ds(2 * HIDDEN + j * L, L)]
        bi = par_v[pl.ds(3 * HIDDEN + j * L, L)]
        acc_a = acc_a + w * w1
        acc_b = acc_b + w * w2
        acc_c = acc_c + bi * (w1 + w2)
    acc_c = acc_c + par_v[pl.ds(4 * HIDDEN, L)]

    lane = lax.iota(jnp.int32, L)
    _dnums = lax.GatherDimensionNumbers(
        offset_dims=(), collapsed_slice_dims=(0,), start_index_map=(0,))

    def lane_sum(v):
        for sh in (8, 4, 2, 1):
            perm = jnp.bitwise_xor(lane, sh)
            v = v + lax.gather(
                v, perm[:, None], dimension_numbers=_dnums, slice_sizes=(1,),
                mode=lax.GatherScatterMode.PROMISE_IN_BOUNDS)
        return v

    a = lane_sum(acc_a)
    b = lane_sum(acc_b)
    c = lane_sum(acc_c)

    d_s.wait()

    @plsc.parallel_loop(0, EPT, L, unroll=U)
    def _(i):
        vs = plsc.load_gather(s_v, [src_v[pl.ds(i, L)]])
        vd = plsc.load_gather(s_v, [dst_v[pl.ds(i, L)]])
        out_v[pl.ds(i, L)] = a * vs + b * vd + c

    pltpu.sync_copy(out_v, out_hbm.at[pl.ds(s_id * EPT, EPT)])


def kernel(x, edge_index, W_lin, bias, W_dec, b_dec):
    ei = edge_index.astype(jnp.int32)

    params = jnp.concatenate([
        W_lin[0],
        W_dec[:, 0],
        bias,
        jnp.pad(b_dec, (0, L - 1)),
    ])

    return _sc_kernel(x[:, 0], ei, params).reshape(N_EDGES, 1)
